# baseline (speedup 1.0000x reference)
import jax
import jax.numpy as jnp
from jax import lax
from jax.experimental import pallas as pl
from jax.experimental.pallas import tpu as pltpu
from jax.experimental.pallas import tpu_sc as plsc

N_LEVELS = 4
F_DIM = 32
BATCH = 32768
NPIX = [12 * (2 ** i) ** 2 for i in range(N_LEVELS)]
ROW_OFF = [0, 12, 60, 252]
TOT_ROWS = 1020

NC, NS = 2, 16
NW = NC * NS
PTS_PER_TILE = BATCH // NW
CHUNK = 256
NCHUNK = PTS_PER_TILE // CHUNK
NGRP = CHUNK // 16
IDX_BLK = 9 * N_LEVELS * CHUNK
LL_BLK = 2 * 9 * N_LEVELS * CHUNK
LVL = 9 * CHUNK
ROW_W = F_DIM // 2


def _splat(v, i):
    idx = jnp.full((16,), i, jnp.int32)
    return lax.gather(
        v, idx[:, None],
        lax.GatherDimensionNumbers(offset_dims=(), collapsed_slice_dims=(0,),
                                   start_index_map=(0,)),
        slice_sizes=(1,),
        mode=lax.GatherScatterMode.PROMISE_IN_BOUNDS)


def _rsqrt(s):
    i = lax.bitcast_convert_type(s, jnp.int32)
    i = jnp.int32(0x5F3759DF) - lax.shift_right_logical(i, 1)
    y = lax.bitcast_convert_type(i, jnp.float32)
    for _ in range(3):
        y = y * (1.5 - 0.5 * s * y * y)
    return y


def _body(table_f, idx_f, pll_f, nll_f, out_hbm,
          table_v, idx_v, ll_v, out_v, sem):
    wid = lax.axis_index("s") * NC + lax.axis_index("c")

    pltpu.sync_copy(table_f, table_v)

    io = lax.broadcasted_iota(jnp.int32, (16,), 0)
    io8 = 8 * io

    def chunk_body(ci, carry):
        c = wid * NCHUNK + ci
        cbase = c * CHUNK
        descs = [pltpu.make_async_copy(
            idx_f.at[pl.ds(c * IDX_BLK, IDX_BLK)], idx_v, sem)]
        for l in range(N_LEVELS):
            for ch in range(2):
                dst0 = ch * LVL * N_LEVELS + l * LVL
                descs.append(pltpu.make_async_copy(
                    pll_f.at[pl.ds((l * 2 + ch) * BATCH + cbase, CHUNK)],
                    ll_v.at[pl.ds(dst0, CHUNK)], sem))
                for j in range(8):
                    descs.append(pltpu.make_async_copy(
                        nll_f.at[pl.ds(((l * 2 + ch) * 8 + j) * BATCH + cbase,
                                       CHUNK)],
                        ll_v.at[pl.ds(dst0 + (j + 1) * CHUNK, CHUNK)], sem))
        for d in descs:
            d.start()
        for d in descs:
            d.wait()

        def level_body(l, carry2):
            roff = (lax.shift_left(jnp.int32(1), 2 * l) - 1) * 4
            lbase = l * LVL

            def group_body(g, carry3):
                p0 = g * 16
                pvec = idx_v[pl.ds(lbase + p0, 16)]
                crow = (pvec + roff) * ROW_W
                plat = ll_v[pl.ds(lbase + p0, 16)]
                plon = ll_v[pl.ds(LVL * N_LEVELS + lbase + p0, 16)]

                ws = []
                rowbases = []
                for j in range(8):
                    nb = lbase + (j + 1) * CHUNK + p0
                    nj = idx_v[pl.ds(nb, 16)]
                    rowbases.append((jnp.maximum(nj, 0) + roff) * ROW_W)
                    nlat = ll_v[pl.ds(nb, 16)]
                    nlon = ll_v[pl.ds(LVL * N_LEVELS + nb, 16)]
                    dlat = nlat - plat
                    dlon = nlon - plon
                    w = _rsqrt(dlat * dlat + dlon * dlon)
                    ws.append(jnp.where(nj >= 0, w, 0.0))

                col_e = io8 + l
                col_o = col_e + 4

                @plsc.parallel_loop(0, 16, unroll=4)
                def p_loop(p):
                    def halves(gw):
                        ev = lax.bitcast_convert_type(
                            lax.shift_left(gw, 16), jnp.float32)
                        od = lax.bitcast_convert_type(
                            gw & jnp.int32(-65536), jnp.float32)
                        return ev, od

                    cb = _splat(crow, p)
                    a0, a1 = halves(plsc.load_gather(table_v, [cb + io]))
                    t0s, t1s = [], []
                    for j in range(8):
                        rb = _splat(rowbases[j], p)
                        wv = _splat(ws[j], p)
                        ev, od = halves(
                            plsc.load_gather(table_v, [rb + io]))
                        t0s.append(ev * wv)
                        t1s.append(od * wv)

                    def red(cc, t):
                        return ((cc + ((t[0] + t[1]) + (t[2] + t[3])))
                                + ((t[4] + t[5]) + (t[6] + t[7])))

                    rowp = jnp.full((16,), p0 + p, jnp.int32)
                    plsc.store_scatter(out_v, [rowp, col_e], red(a0, t0s))
                    plsc.store_scatter(out_v, [rowp, col_o], red(a1, t1s))

                return carry3

            return lax.fori_loop(0, NGRP, group_body, carry2)

        lax.fori_loop(0, N_LEVELS, level_body, 0)
        pltpu.sync_copy(out_v, out_hbm.at[pl.ds(cbase, CHUNK), :])
        return carry

    lax.fori_loop(0, NCHUNK, chunk_body, 0)


@jax.jit
def _heal_encoding_sc(params, pix_idx, neigh_idx, pix_ll, neigh_ll):
    table = jnp.concatenate(
        [params[l, :NPIX[l], :] for l in range(N_LEVELS)], axis=0)
    table = lax.bitcast_convert_type(
        table.astype(jnp.bfloat16).reshape(TOT_ROWS * ROW_W, 2),
        jnp.int32).reshape(-1)
    idx_blk = jnp.concatenate(
        [pix_idx.reshape(N_LEVELS, 1, 128, CHUNK),
         neigh_idx.reshape(N_LEVELS, 8, 128, CHUNK)], axis=1)
    idx_p = idx_blk.transpose(2, 0, 1, 3).reshape(-1)
    pll_t = pix_ll.transpose(0, 2, 1).reshape(-1)
    nll_t = neigh_ll.transpose(0, 2, 1).reshape(-1)

    mesh = plsc.VectorSubcoreMesh(core_axis_name="c", subcore_axis_name="s")
    kfn = pl.kernel(
        _body,
        out_type=jax.ShapeDtypeStruct((BATCH, N_LEVELS * F_DIM), jnp.float32),
        mesh=mesh,
        scratch_types=[
            pltpu.VMEM((TOT_ROWS * ROW_W,), jnp.int32),
            pltpu.VMEM((IDX_BLK,), jnp.int32),
            pltpu.VMEM((LL_BLK,), jnp.float32),
            pltpu.VMEM((CHUNK, N_LEVELS * F_DIM), jnp.float32),
            pltpu.SemaphoreType.DMA,
        ],
        compiler_params=pltpu.CompilerParams(needs_layout_passes=False),
    )
    return kfn(table, idx_p, pll_t, nll_t)


def kernel(params, all_level_pixel_index, all_level_neigh_index,
           all_level_pixel_latlon, all_level_neigh_latlon):
    return _heal_encoding_sc(
        params,
        all_level_pixel_index.astype(jnp.int32),
        all_level_neigh_index.astype(jnp.int32),
        all_level_pixel_latlon,
        all_level_neigh_latlon,
    )


# speedup vs baseline: 1.2682x; 1.2682x over previous
import jax
import jax.numpy as jnp
from jax import lax
from jax.experimental import pallas as pl
from jax.experimental.pallas import tpu as pltpu
from jax.experimental.pallas import tpu_sc as plsc

N_LEVELS = 4
F_DIM = 32
BATCH = 32768
NPIX = [12 * (2 ** i) ** 2 for i in range(N_LEVELS)]
ROW_OFF = [0, 12, 60, 252]
TOT_ROWS = 1020

NC, NS = 2, 16
NW = NC * NS
PTS_PER_TILE = BATCH // NW
CHUNK = 256
NCHUNK = PTS_PER_TILE // CHUNK
NGRP = CHUNK // 16
IDX_BLK = 9 * N_LEVELS * CHUNK
LL_BLK = 2 * 9 * N_LEVELS * CHUNK
LVL = 9 * CHUNK
ROW_W = F_DIM // 2


def _splat(v, i):
    idx = jnp.full((16,), i, jnp.int32)
    return lax.gather(
        v, idx[:, None],
        lax.GatherDimensionNumbers(offset_dims=(), collapsed_slice_dims=(0,),
                                   start_index_map=(0,)),
        slice_sizes=(1,),
        mode=lax.GatherScatterMode.PROMISE_IN_BOUNDS)


def _rsqrt(s):
    i = lax.bitcast_convert_type(s, jnp.int32)
    i = jnp.int32(0x5F3759DF) - lax.shift_right_logical(i, 1)
    y = lax.bitcast_convert_type(i, jnp.float32)
    for _ in range(3):
        y = y * (1.5 - 0.5 * s * y * y)
    return y


def _body(table_f, idx_f, pll_f, nll_f, out_hbm,
          table_v, idx_v, ll_v, out_v, sem):
    wid = lax.axis_index("s") * NC + lax.axis_index("c")

    pltpu.sync_copy(table_f, table_v)

    io = lax.broadcasted_iota(jnp.int32, (16,), 0)
    io8 = 8 * io

    def chunk_body(ci, carry):
        c = wid * NCHUNK + ci
        cbase = c * CHUNK
        descs = [pltpu.make_async_copy(
            idx_f.at[pl.ds(c * IDX_BLK, IDX_BLK)], idx_v, sem)]
        for l in range(N_LEVELS):
            for ch in range(2):
                dst0 = ch * LVL * N_LEVELS + l * LVL
                descs.append(pltpu.make_async_copy(
                    pll_f.at[pl.ds((l * 2 + ch) * BATCH + cbase, CHUNK)],
                    ll_v.at[pl.ds(dst0, CHUNK)], sem))
                for j in range(8):
                    descs.append(pltpu.make_async_copy(
                        nll_f.at[pl.ds(((l * 2 + ch) * 8 + j) * BATCH + cbase,
                                       CHUNK)],
                        ll_v.at[pl.ds(dst0 + (j + 1) * CHUNK, CHUNK)], sem))
        for d in descs:
            d.start()
        for d in descs:
            d.wait()

        def level_body(l, carry2):
            roff = (lax.shift_left(jnp.int32(1), 2 * l) - 1) * 4
            lbase = l * LVL

            def group_body(g, carry3):
                p0 = g * 16
                pvec = idx_v[pl.ds(lbase + p0, 16)]
                crow = (pvec + roff) * ROW_W
                plat = ll_v[pl.ds(lbase + p0, 16)]
                plon = ll_v[pl.ds(LVL * N_LEVELS + lbase + p0, 16)]

                w2s = []
                rowbases = []
                for j in range(8):
                    nb = lbase + (j + 1) * CHUNK + p0
                    nj = idx_v[pl.ds(nb, 16)]
                    rowbases.append((jnp.maximum(nj, 0) + roff) * ROW_W)
                    nlat = ll_v[pl.ds(nb, 16)]
                    nlon = ll_v[pl.ds(LVL * N_LEVELS + nb, 16)]
                    dlat = nlat - plat
                    dlon = nlon - plon
                    w = _rsqrt(dlat * dlat + dlon * dlon)
                    w = jnp.where(nj >= 0, w, 0.0)
                    wb = lax.shift_right_logical(
                        lax.bitcast_convert_type(w, jnp.int32)
                        + jnp.int32(0x8000), 16)
                    w2s.append(lax.shift_left(wb, 16) | wb)

                col_e = io8 + l
                col_o = col_e + 4

                @plsc.parallel_loop(0, 16, unroll=2)
                def p_loop(p):
                    cb = _splat(crow, p)
                    c32 = plsc.load_gather(table_v, [cb + io])
                    acc = plsc.bitcast(c32, jnp.bfloat16)
                    ts = []
                    for j in range(8):
                        rb = _splat(rowbases[j], p)
                        wv = plsc.bitcast(_splat(w2s[j], p), jnp.bfloat16)
                        g = plsc.bitcast(
                            plsc.load_gather(table_v, [rb + io]),
                            jnp.bfloat16)
                        ts.append(g * wv)
                    acc = ((acc + ((ts[0] + ts[1]) + (ts[2] + ts[3])))
                           + ((ts[4] + ts[5]) + (ts[6] + ts[7])))
                    aw = plsc.bitcast(acc, jnp.int32)
                    ev = lax.bitcast_convert_type(
                        lax.shift_left(aw, 16), jnp.float32)
                    od = lax.bitcast_convert_type(
                        aw & jnp.int32(-65536), jnp.float32)
                    rowp = jnp.full((16,), p0 + p, jnp.int32)
                    plsc.store_scatter(out_v, [rowp, col_e], ev)
                    plsc.store_scatter(out_v, [rowp, col_o], od)

                return carry3

            return lax.fori_loop(0, NGRP, group_body, carry2)

        lax.fori_loop(0, N_LEVELS, level_body, 0)
        pltpu.sync_copy(out_v, out_hbm.at[pl.ds(cbase, CHUNK), :])
        return carry

    lax.fori_loop(0, NCHUNK, chunk_body, 0)


@jax.jit
def _heal_encoding_sc(params, pix_idx, neigh_idx, pix_ll, neigh_ll):
    table = jnp.concatenate(
        [params[l, :NPIX[l], :] for l in range(N_LEVELS)], axis=0)
    table = lax.bitcast_convert_type(
        table.astype(jnp.bfloat16).reshape(TOT_ROWS * ROW_W, 2),
        jnp.int32).reshape(-1)
    idx_blk = jnp.concatenate(
        [pix_idx.reshape(N_LEVELS, 1, 128, CHUNK),
         neigh_idx.reshape(N_LEVELS, 8, 128, CHUNK)], axis=1)
    idx_p = idx_blk.transpose(2, 0, 1, 3).reshape(-1)
    pll_t = pix_ll.transpose(0, 2, 1).reshape(-1)
    nll_t = neigh_ll.transpose(0, 2, 1).reshape(-1)

    mesh = plsc.VectorSubcoreMesh(core_axis_name="c", subcore_axis_name="s")
    kfn = pl.kernel(
        _body,
        out_type=jax.ShapeDtypeStruct((BATCH, N_LEVELS * F_DIM), jnp.float32),
        mesh=mesh,
        scratch_types=[
            pltpu.VMEM((TOT_ROWS * ROW_W,), jnp.int32),
            pltpu.VMEM((IDX_BLK,), jnp.int32),
            pltpu.VMEM((LL_BLK,), jnp.float32),
            pltpu.VMEM((CHUNK, N_LEVELS * F_DIM), jnp.float32),
            pltpu.SemaphoreType.DMA,
        ],
        compiler_params=pltpu.CompilerParams(needs_layout_passes=False),
    )
    return kfn(table, idx_p, pll_t, nll_t)


def kernel(params, all_level_pixel_index, all_level_neigh_index,
           all_level_pixel_latlon, all_level_neigh_latlon):
    return _heal_encoding_sc(
        params,
        all_level_pixel_index.astype(jnp.int32),
        all_level_neigh_index.astype(jnp.int32),
        all_level_pixel_latlon,
        all_level_neigh_latlon,
    )


# speedup vs baseline: 1.3692x; 1.0797x over previous
import jax
import jax.numpy as jnp
from jax import lax
from jax.experimental import pallas as pl
from jax.experimental.pallas import tpu as pltpu
from jax.experimental.pallas import tpu_sc as plsc

N_LEVELS = 4
F_DIM = 32
BATCH = 32768
NPIX = [12 * (2 ** i) ** 2 for i in range(N_LEVELS)]
ROW_OFF = [0, 12, 60, 252]
TOT_ROWS = 1020

NC, NS = 2, 16
NW = NC * NS
PTS_PER_TILE = BATCH // NW
CHUNK = 256
NCHUNK = PTS_PER_TILE // CHUNK
NGRP = CHUNK // 16
IDX_BLK = 9 * N_LEVELS * CHUNK
LL_BLK = 2 * 9 * N_LEVELS * CHUNK
LVL = 9 * CHUNK
ROW_W = F_DIM // 2


def _splat(v, i):
    idx = jnp.full((16,), i, jnp.int32)
    return lax.gather(
        v, idx[:, None],
        lax.GatherDimensionNumbers(offset_dims=(), collapsed_slice_dims=(0,),
                                   start_index_map=(0,)),
        slice_sizes=(1,),
        mode=lax.GatherScatterMode.PROMISE_IN_BOUNDS)


def _rsqrt(s):
    i = lax.bitcast_convert_type(s, jnp.int32)
    i = jnp.int32(0x5F3759DF) - lax.shift_right_logical(i, 1)
    y = lax.bitcast_convert_type(i, jnp.float32)
    for _ in range(3):
        y = y * (1.5 - 0.5 * s * y * y)
    return y


def _body(pix_f, nidx_f, table_f, pll_f, nll_f, out_hbm,
          table_v, idx_v0, ll_v0, idx_v1, ll_v1, out_v, sem, osem):
    wid = lax.axis_index("s") * NC + lax.axis_index("c")

    pltpu.sync_copy(table_f, table_v)

    io = lax.broadcasted_iota(jnp.int32, (16,), 0)
    io8 = 8 * io
    bufs = [(idx_v0, ll_v0), (idx_v1, ll_v1)]

    def issue(ci):
        c = wid * NCHUNK + ci
        cbase = c * CHUNK
        idx_v, ll_v = bufs[ci & 1]
        descs = []
        for l in range(N_LEVELS):
            descs.append(pltpu.make_async_copy(
                pix_f.at[pl.ds(l * BATCH + cbase, CHUNK)],
                idx_v.at[pl.ds(l * LVL, CHUNK)], sem))
            for j in range(8):
                descs.append(pltpu.make_async_copy(
                    nidx_f.at[pl.ds((l * 8 + j) * BATCH + cbase, CHUNK)],
                    idx_v.at[pl.ds(l * LVL + (j + 1) * CHUNK, CHUNK)], sem))
            for ch in range(2):
                dst0 = ch * LVL * N_LEVELS + l * LVL
                descs.append(pltpu.make_async_copy(
                    pll_f.at[pl.ds((l * 2 + ch) * BATCH + cbase, CHUNK)],
                    ll_v.at[pl.ds(dst0, CHUNK)], sem))
                for j in range(8):
                    descs.append(pltpu.make_async_copy(
                        nll_f.at[pl.ds(((l * 2 + ch) * 8 + j) * BATCH + cbase,
                                       CHUNK)],
                        ll_v.at[pl.ds(dst0 + (j + 1) * CHUNK, CHUNK)], sem))
        for d in descs:
            d.start()
        return descs

    def compute_chunk(idx_v, ll_v):
        def level_body(l, carry2):
            roff = (lax.shift_left(jnp.int32(1), 2 * l) - 1) * 4
            lbase = l * LVL

            def group_body(g, carry3):
                p0 = g * 16
                pvec = idx_v[pl.ds(lbase + p0, 16)]
                crow = (pvec + roff) * ROW_W
                plat = ll_v[pl.ds(lbase + p0, 16)]
                plon = ll_v[pl.ds(LVL * N_LEVELS + lbase + p0, 16)]

                w2s = []
                rowbases = []
                for j in range(8):
                    nb = lbase + (j + 1) * CHUNK + p0
                    nj = idx_v[pl.ds(nb, 16)]
                    rowbases.append((jnp.maximum(nj, 0) + roff) * ROW_W)
                    nlat = ll_v[pl.ds(nb, 16)]
                    nlon = ll_v[pl.ds(LVL * N_LEVELS + nb, 16)]
                    dlat = nlat - plat
                    dlon = nlon - plon
                    w = _rsqrt(dlat * dlat + dlon * dlon)
                    w = jnp.where(nj >= 0, w, 0.0)
                    wb = lax.shift_right_logical(
                        lax.bitcast_convert_type(w, jnp.int32)
                        + jnp.int32(0x8000), 16)
                    w2s.append(lax.shift_left(wb, 16) | wb)

                col_e = io8 + l
                col_o = col_e + 4

                @plsc.parallel_loop(0, 16, unroll=2)
                def p_loop(p):
                    cb = _splat(crow, p)
                    c32 = plsc.load_gather(table_v, [cb + io])
                    acc = plsc.bitcast(c32, jnp.bfloat16)
                    ts = []
                    for j in range(8):
                        rb = _splat(rowbases[j], p)
                        wv = plsc.bitcast(_splat(w2s[j], p), jnp.bfloat16)
                        g = plsc.bitcast(
                            plsc.load_gather(table_v, [rb + io]),
                            jnp.bfloat16)
                        ts.append(g * wv)
                    acc = ((acc + ((ts[0] + ts[1]) + (ts[2] + ts[3])))
                           + ((ts[4] + ts[5]) + (ts[6] + ts[7])))
                    aw = plsc.bitcast(acc, jnp.int32)
                    ev = lax.bitcast_convert_type(
                        lax.shift_left(aw, 16), jnp.float32)
                    od = lax.bitcast_convert_type(
                        aw & jnp.int32(-65536), jnp.float32)
                    rowp = jnp.full((16,), p0 + p, jnp.int32)
                    plsc.store_scatter(out_v, [rowp, col_e], ev)
                    plsc.store_scatter(out_v, [rowp, col_o], od)

                return carry3

            return lax.fori_loop(0, NGRP, group_body, carry2)

        lax.fori_loop(0, N_LEVELS, level_body, 0)

    pend = issue(0)
    odesc = None
    for ci in range(NCHUNK):
        for d in pend:
            d.wait()
        if ci + 1 < NCHUNK:
            pend = issue(ci + 1)
        if odesc is not None:
            odesc.wait()
        idx_v, ll_v = bufs[ci & 1]
        compute_chunk(idx_v, ll_v)
        cbase = (wid * NCHUNK + ci) * CHUNK
        odesc = pltpu.make_async_copy(
            out_v, out_hbm.at[pl.ds(cbase, CHUNK), :], osem)
        odesc.start()
    odesc.wait()


@jax.jit
def _heal_encoding_sc(params, pix_idx, neigh_idx, pix_ll, neigh_ll):
    table = jnp.concatenate(
        [params[l, :NPIX[l], :] for l in range(N_LEVELS)], axis=0)
    table = lax.bitcast_convert_type(
        table.astype(jnp.bfloat16).reshape(TOT_ROWS * ROW_W, 2),
        jnp.int32).reshape(-1)
    pix_f = pix_idx.reshape(-1)
    nidx_f = neigh_idx.reshape(-1)
    pll_t = pix_ll.transpose(0, 2, 1).reshape(-1)
    nll_t = neigh_ll.transpose(0, 2, 1).reshape(-1)

    mesh = plsc.VectorSubcoreMesh(core_axis_name="c", subcore_axis_name="s")
    kfn = pl.kernel(
        _body,
        out_type=jax.ShapeDtypeStruct((BATCH, N_LEVELS * F_DIM), jnp.float32),
        mesh=mesh,
        scratch_types=[
            pltpu.VMEM((TOT_ROWS * ROW_W,), jnp.int32),
            pltpu.VMEM((IDX_BLK,), jnp.int32),
            pltpu.VMEM((LL_BLK,), jnp.float32),
            pltpu.VMEM((IDX_BLK,), jnp.int32),
            pltpu.VMEM((LL_BLK,), jnp.float32),
            pltpu.VMEM((CHUNK, N_LEVELS * F_DIM), jnp.float32),
            pltpu.SemaphoreType.DMA,
            pltpu.SemaphoreType.DMA,
        ],
        compiler_params=pltpu.CompilerParams(needs_layout_passes=False),
    )
    return kfn(pix_f, nidx_f, table, pll_t, nll_t)


def kernel(params, all_level_pixel_index, all_level_neigh_index,
           all_level_pixel_latlon, all_level_neigh_latlon):
    return _heal_encoding_sc(
        params,
        all_level_pixel_index.astype(jnp.int32),
        all_level_neigh_index.astype(jnp.int32),
        all_level_pixel_latlon,
        all_level_neigh_latlon,
    )


# speedup vs baseline: 1.4003x; 1.0227x over previous
import jax
import jax.numpy as jnp
from jax import lax
from jax.experimental import pallas as pl
from jax.experimental.pallas import tpu as pltpu
from jax.experimental.pallas import tpu_sc as plsc

N_LEVELS = 4
F_DIM = 32
BATCH = 32768
NPIX = [12 * (2 ** i) ** 2 for i in range(N_LEVELS)]
ROW_OFF = [0, 12, 60, 252]
TOT_ROWS = 1020

NC, NS = 2, 16
NW = NC * NS
PTS_PER_TILE = BATCH // NW
CHUNK = 256
NCHUNK = PTS_PER_TILE // CHUNK
NGRP = CHUNK // 16
IDX_BLK = 9 * N_LEVELS * CHUNK
LL_BLK = 2 * 9 * N_LEVELS * CHUNK
LVL = 9 * CHUNK
ROW_W = F_DIM // 2


def _splat(v, i):
    idx = jnp.full((16,), i, jnp.int32)
    return lax.gather(
        v, idx[:, None],
        lax.GatherDimensionNumbers(offset_dims=(), collapsed_slice_dims=(0,),
                                   start_index_map=(0,)),
        slice_sizes=(1,),
        mode=lax.GatherScatterMode.PROMISE_IN_BOUNDS)


def _rsqrt(s):
    i = lax.bitcast_convert_type(s, jnp.int32)
    i = jnp.int32(0x5F3759DF) - lax.shift_right_logical(i, 1)
    y = lax.bitcast_convert_type(i, jnp.float32)
    for _ in range(2):
        y = y * (1.5 - 0.5 * s * y * y)
    return y


def _body(pix_f, nidx_f, table_f, pll_f, nll_f, out_hbm,
          table_v, idx_v0, ll_v0, idx_v1, ll_v1, out_v, sem, osem):
    wid = lax.axis_index("s") * NC + lax.axis_index("c")

    pltpu.sync_copy(table_f, table_v)

    io = lax.broadcasted_iota(jnp.int32, (16,), 0)
    io8 = 8 * io
    bufs = [(idx_v0, ll_v0), (idx_v1, ll_v1)]

    def issue(ci):
        c = wid * NCHUNK + ci
        cbase = c * CHUNK
        idx_v, ll_v = bufs[ci & 1]
        descs = []
        for l in range(N_LEVELS):
            descs.append(pltpu.make_async_copy(
                pix_f.at[pl.ds(l * BATCH + cbase, CHUNK)],
                idx_v.at[pl.ds(l * LVL, CHUNK)], sem))
            for j in range(8):
                descs.append(pltpu.make_async_copy(
                    nidx_f.at[pl.ds((l * 8 + j) * BATCH + cbase, CHUNK)],
                    idx_v.at[pl.ds(l * LVL + (j + 1) * CHUNK, CHUNK)], sem))
            for ch in range(2):
                dst0 = ch * LVL * N_LEVELS + l * LVL
                descs.append(pltpu.make_async_copy(
                    pll_f.at[pl.ds((l * 2 + ch) * BATCH + cbase, CHUNK)],
                    ll_v.at[pl.ds(dst0, CHUNK)], sem))
                for j in range(8):
                    descs.append(pltpu.make_async_copy(
                        nll_f.at[pl.ds(((l * 2 + ch) * 8 + j) * BATCH + cbase,
                                       CHUNK)],
                        ll_v.at[pl.ds(dst0 + (j + 1) * CHUNK, CHUNK)], sem))
        for d in descs:
            d.start()
        return descs

    def compute_chunk(idx_v, ll_v):
        def level_body(l, carry2):
            roff = (lax.shift_left(jnp.int32(1), 2 * l) - 1) * 4
            lbase = l * LVL

            def group_body(g, carry3):
                p0 = g * 16
                pvec = idx_v[pl.ds(lbase + p0, 16)]
                crow = (pvec + roff) * ROW_W
                plat = ll_v[pl.ds(lbase + p0, 16)]
                plon = ll_v[pl.ds(LVL * N_LEVELS + lbase + p0, 16)]

                w2s = []
                rowbases = []
                for j in range(8):
                    nb = lbase + (j + 1) * CHUNK + p0
                    nj = idx_v[pl.ds(nb, 16)]
                    rowbases.append((jnp.maximum(nj, 0) + roff) * ROW_W)
                    nlat = ll_v[pl.ds(nb, 16)]
                    nlon = ll_v[pl.ds(LVL * N_LEVELS + nb, 16)]
                    dlat = nlat - plat
                    dlon = nlon - plon
                    w = _rsqrt(dlat * dlat + dlon * dlon)
                    w = jnp.where(nj >= 0, w, 0.0)
                    wb = lax.shift_right_logical(
                        lax.bitcast_convert_type(w, jnp.int32)
                        + jnp.int32(0x8000), 16)
                    w2s.append(lax.shift_left(wb, 16) | wb)

                col_e = io8 + l
                col_o = col_e + 4

                @plsc.parallel_loop(0, 16, unroll=2)
                def p_loop(p):
                    cb = _splat(crow, p)
                    c32 = plsc.load_gather(table_v, [cb + io])
                    acc = plsc.bitcast(c32, jnp.bfloat16)
                    ts = []
                    for j in range(8):
                        rb = _splat(rowbases[j], p)
                        wv = plsc.bitcast(_splat(w2s[j], p), jnp.bfloat16)
                        g = plsc.bitcast(
                            plsc.load_gather(table_v, [rb + io]),
                            jnp.bfloat16)
                        ts.append(g * wv)
                    acc = ((acc + ((ts[0] + ts[1]) + (ts[2] + ts[3])))
                           + ((ts[4] + ts[5]) + (ts[6] + ts[7])))
                    aw = plsc.bitcast(acc, jnp.int32)
                    ev = lax.bitcast_convert_type(
                        lax.shift_left(aw, 16), jnp.float32)
                    od = lax.bitcast_convert_type(
                        aw & jnp.int32(-65536), jnp.float32)
                    rowp = jnp.full((16,), p0 + p, jnp.int32)
                    plsc.store_scatter(out_v, [rowp, col_e], ev)
                    plsc.store_scatter(out_v, [rowp, col_o], od)

                return carry3

            return lax.fori_loop(0, NGRP, group_body, carry2)

        lax.fori_loop(0, N_LEVELS, level_body, 0)

    pend = issue(0)
    odesc = None
    for ci in range(NCHUNK):
        for d in pend:
            d.wait()
        if ci + 1 < NCHUNK:
            pend = issue(ci + 1)
        if odesc is not None:
            odesc.wait()
        idx_v, ll_v = bufs[ci & 1]
        compute_chunk(idx_v, ll_v)
        cbase = (wid * NCHUNK + ci) * CHUNK
        odesc = pltpu.make_async_copy(
            out_v, out_hbm.at[pl.ds(cbase, CHUNK), :], osem)
        odesc.start()
    odesc.wait()


@jax.jit
def _heal_encoding_sc(params, pix_idx, neigh_idx, pix_ll, neigh_ll):
    table = jnp.concatenate(
        [params[l, :NPIX[l], :] for l in range(N_LEVELS)], axis=0)
    table = lax.bitcast_convert_type(
        table.astype(jnp.bfloat16).reshape(TOT_ROWS * ROW_W, 2),
        jnp.int32).reshape(-1)
    pix_f = pix_idx.reshape(-1)
    nidx_f = neigh_idx.reshape(-1)
    pll_t = pix_ll.transpose(0, 2, 1).reshape(-1)
    nll_t = neigh_ll.transpose(0, 2, 1).reshape(-1)

    mesh = plsc.VectorSubcoreMesh(core_axis_name="c", subcore_axis_name="s")
    kfn = pl.kernel(
        _body,
        out_type=jax.ShapeDtypeStruct((BATCH, N_LEVELS * F_DIM), jnp.float32),
        mesh=mesh,
        scratch_types=[
            pltpu.VMEM((TOT_ROWS * ROW_W,), jnp.int32),
            pltpu.VMEM((IDX_BLK,), jnp.int32),
            pltpu.VMEM((LL_BLK,), jnp.float32),
            pltpu.VMEM((IDX_BLK,), jnp.int32),
            pltpu.VMEM((LL_BLK,), jnp.float32),
            pltpu.VMEM((CHUNK, N_LEVELS * F_DIM), jnp.float32),
            pltpu.SemaphoreType.DMA,
            pltpu.SemaphoreType.DMA,
        ],
        compiler_params=pltpu.CompilerParams(needs_layout_passes=False),
    )
    return kfn(pix_f, nidx_f, table, pll_t, nll_t)


def kernel(params, all_level_pixel_index, all_level_neigh_index,
           all_level_pixel_latlon, all_level_neigh_latlon):
    return _heal_encoding_sc(
        params,
        all_level_pixel_index.astype(jnp.int32),
        all_level_neigh_index.astype(jnp.int32),
        all_level_pixel_latlon,
        all_level_neigh_latlon,
    )
